# Initial kernel scaffold; baseline (speedup 1.0000x reference)
import jax
import jax.numpy as jnp
from jax.experimental import pallas as pl


def kernel(x, edge_index, W, b):
    raise NotImplementedError("write your pallas kernel here")


# speedup vs baseline: 13.2487x; 13.2487x over previous
import functools

import jax
import jax.numpy as jnp
from jax import lax
from jax.experimental import pallas as pl
from jax.experimental.pallas import tpu as pltpu
from jax.experimental.pallas import tpu_sc as plsc

N = 10000
D = 128
NC, NS, L = 2, 16, 16
NW = NC * NS
CHUNK = 128
NCHUNK = 80
EPW = CHUNK * NCHUNK
EPAD = EPW * NW
ROWS_PER_TILE = 632
NPAD = ROWS_PER_TILE * NS
HIST_PER_TILE = 640
HIST_PAD = HIST_PER_TILE * NS

_MESH = plsc.VectorSubcoreMesh(core_axis_name="c", subcore_axis_name="s")


@functools.partial(
    pl.kernel,
    out_type=jax.ShapeDtypeStruct((NC, HIST_PAD), jnp.float32),
    mesh=_MESH,
    scratch_types=[
        pltpu.VMEM((NCHUNK, CHUNK), jnp.int32),
        pltpu.VMEM((CHUNK,), jnp.float32),
        pltpu.VMEM((HIST_PER_TILE,), jnp.float32),
        pltpu.VMEM_SHARED((HIST_PAD,), jnp.float32),
    ],
)
def _sc_hist(dst_hbm, out_hbm, idx_v, ones_v, zeros_v, hist_sh):
    c = lax.axis_index("c")
    s = lax.axis_index("s")
    w = s * NC + c

    def fill_ones(i, carry):
        ones_v[pl.ds(i * L, L)] = jnp.full((L,), 1.0, jnp.float32)
        return carry

    lax.fori_loop(0, CHUNK // L, fill_ones, 0)

    def fill_zeros(i, carry):
        zeros_v[pl.ds(i * L, L)] = jnp.zeros((L,), jnp.float32)
        return carry

    lax.fori_loop(0, HIST_PER_TILE // L, fill_zeros, 0)

    pltpu.sync_copy(zeros_v, hist_sh.at[pl.ds(s * HIST_PER_TILE, HIST_PER_TILE)])
    pltpu.sync_copy(dst_hbm.at[w], idx_v)
    plsc.subcore_barrier()

    def body(j, carry):
        pltpu.sync_copy(ones_v, hist_sh.at[idx_v.at[j]], add=True)
        return carry

    lax.fori_loop(0, NCHUNK, body, 0)
    plsc.subcore_barrier()

    @pl.when(s == 0)
    def _():
        pltpu.sync_copy(hist_sh, out_hbm.at[c])


def _tc_transform_body(x_ref, w_ref, hist_ref, hs_ref, dis_ref):
    deg = hist_ref[0, :N] + hist_ref[1, :N] + 1.0
    dis = lax.rsqrt(deg)
    h = jnp.dot(x_ref[...], w_ref[...], preferred_element_type=jnp.float32)
    hs_ref[...] = h * dis[:, None]
    dis_ref[...] = dis


_tc_transform = pl.pallas_call(
    _tc_transform_body,
    out_shape=[
        jax.ShapeDtypeStruct((N, D), jnp.float32),
        jax.ShapeDtypeStruct((N,), jnp.float32),
    ],
)


HALF = NCHUNK // 2


@functools.partial(
    pl.kernel,
    out_type=jax.ShapeDtypeStruct((NC, NPAD, D), jnp.float32),
    mesh=_MESH,
    scratch_types=[
        pltpu.VMEM((HALF, CHUNK), jnp.int32),
        pltpu.VMEM((HALF, CHUNK), jnp.int32),
        pltpu.VMEM((CHUNK, D), jnp.float32),
        pltpu.VMEM((CHUNK, D), jnp.float32),
        pltpu.VMEM_SHARED((NPAD, D), jnp.float32),
        pltpu.SemaphoreType.DMA,
        pltpu.SemaphoreType.DMA,
    ],
)
def _sc_agg(hs_hbm, srcw_hbm, dstw_hbm, zeros_hbm, out_hbm,
            si_v, di_v, buf0, buf1, acc_sh, sem0, sem1):
    c = lax.axis_index("c")
    s = lax.axis_index("s")
    w = s * NC + c

    rslice = pl.ds(s * ROWS_PER_TILE, ROWS_PER_TILE)
    pltpu.sync_copy(zeros_hbm.at[rslice], acc_sh.at[rslice])
    plsc.subcore_barrier()

    for h in range(NCHUNK // HALF):
        hslice = pl.ds(h * HALF, HALF)
        pltpu.sync_copy(srcw_hbm.at[w].at[hslice], si_v)
        pltpu.sync_copy(dstw_hbm.at[w].at[hslice], di_v)

        pltpu.async_copy(hs_hbm.at[si_v.at[0]], buf0, sem0)
        pltpu.async_copy(hs_hbm.at[si_v.at[1]], buf1, sem1)

        def body(jo, carry):
            j = jo * 2
            pltpu.make_async_copy(hs_hbm.at[si_v.at[j]], buf0, sem0).wait()
            pltpu.sync_copy(buf0, acc_sh.at[di_v.at[j]], add=True)
            pltpu.async_copy(hs_hbm.at[si_v.at[j + 2]], buf0, sem0)
            pltpu.make_async_copy(hs_hbm.at[si_v.at[j + 1]], buf1, sem1).wait()
            pltpu.sync_copy(buf1, acc_sh.at[di_v.at[j + 1]], add=True)
            pltpu.async_copy(hs_hbm.at[si_v.at[j + 3]], buf1, sem1)
            return carry

        lax.fori_loop(0, HALF // 2 - 1, body, 0)
        j = HALF - 2
        pltpu.make_async_copy(hs_hbm.at[si_v.at[j]], buf0, sem0).wait()
        pltpu.sync_copy(buf0, acc_sh.at[di_v.at[j]], add=True)
        pltpu.make_async_copy(hs_hbm.at[si_v.at[j + 1]], buf1, sem1).wait()
        pltpu.sync_copy(buf1, acc_sh.at[di_v.at[j + 1]], add=True)

    plsc.subcore_barrier()
    pltpu.sync_copy(acc_sh.at[rslice], out_hbm.at[c].at[rslice])


def _tc_final_body(acc_ref, hs_ref, dis_ref, b_ref, out_ref):
    agg = acc_ref[0, :N, :] + acc_ref[1, :N, :] + hs_ref[...]
    out_ref[...] = jnp.maximum(agg * dis_ref[...][:, None] + b_ref[...][None, :], 0.0)


_tc_final = pl.pallas_call(
    _tc_final_body,
    out_shape=jax.ShapeDtypeStruct((N, D), jnp.float32),
)


def kernel(x, edge_index, W, b):
    src = edge_index[0].astype(jnp.int32)
    dst = edge_index[1].astype(jnp.int32)
    e = src.shape[0]
    pad = EPAD - e
    srcp = jnp.concatenate([src, jnp.zeros((pad,), jnp.int32)]).reshape(NW, NCHUNK, CHUNK)
    dstp = jnp.concatenate([dst, jnp.full((pad,), N, jnp.int32)]).reshape(NW, NCHUNK, CHUNK)
    hist = _sc_hist(dstp)
    hs, dis = _tc_transform(x, W, hist)
    zeros = jnp.zeros((NPAD, D), jnp.float32)
    acc = _sc_agg(hs, srcp, dstp, zeros)
    return _tc_final(acc, hs, dis, b)
